# baseline (speedup 1.0000x reference)
import jax
import jax.numpy as jnp
from jax.experimental import pallas as pl
from jax.experimental.pallas import tpu as pltpu

_E_ROWS = 1_000_000
_DIM = 32
_BLOCK_COLS = 32768
_GRID = (_E_ROWS + _BLOCK_COLS - 1) // _BLOCK_COLS


def _copy_body(in_ref, out_ref):
    out_ref[...] = in_ref[...]


def kernel(embed):
    t = embed.T
    out = pl.pallas_call(
        _copy_body,
        grid=(_GRID,),
        in_specs=[pl.BlockSpec((_DIM, _BLOCK_COLS), lambda i: (0, i))],
        out_specs=pl.BlockSpec((_DIM, _BLOCK_COLS), lambda i: (0, i)),
        out_shape=jax.ShapeDtypeStruct((_DIM, _E_ROWS), jnp.float32),
    )(t)
    return out.T


# speedup vs baseline: 1.0287x; 1.0287x over previous
import jax
import jax.numpy as jnp
from jax.experimental import pallas as pl
from jax.experimental.pallas import tpu as pltpu

_E_ROWS = 1_000_000
_DIM = 32
_BLOCK_COLS = 98304
_GRID = (_E_ROWS + _BLOCK_COLS - 1) // _BLOCK_COLS


def _copy_body(in_ref, out_ref):
    out_ref[...] = in_ref[...]


def kernel(embed):
    t = embed.T
    out = pl.pallas_call(
        _copy_body,
        grid=(_GRID,),
        in_specs=[pl.BlockSpec((_DIM, _BLOCK_COLS), lambda i: (0, i))],
        out_specs=pl.BlockSpec((_DIM, _BLOCK_COLS), lambda i: (0, i)),
        out_shape=jax.ShapeDtypeStruct((_DIM, _E_ROWS), jnp.float32),
    )(t)
    return out.T


# speedup vs baseline: 1.0306x; 1.0019x over previous
import jax
import jax.numpy as jnp
from jax.experimental import pallas as pl
from jax.experimental.pallas import tpu as pltpu

_E_ROWS = 1_000_000
_DIM = 32
_BLOCK_COLS = 106496
_GRID = (_E_ROWS + _BLOCK_COLS - 1) // _BLOCK_COLS


def _copy_body(in_ref, out_ref):
    out_ref[...] = in_ref[...]


def kernel(embed):
    t = embed.T
    out = pl.pallas_call(
        _copy_body,
        grid=(_GRID,),
        in_specs=[pl.BlockSpec((_DIM, _BLOCK_COLS), lambda i: (0, i))],
        out_specs=pl.BlockSpec((_DIM, _BLOCK_COLS), lambda i: (0, i)),
        out_shape=jax.ShapeDtypeStruct((_DIM, _E_ROWS), jnp.float32),
    )(t)
    return out.T


# speedup vs baseline: 1.0307x; 1.0002x over previous
import jax
import jax.numpy as jnp
from jax.experimental import pallas as pl
from jax.experimental.pallas import tpu as pltpu

_E_ROWS = 1_000_000
_DIM = 32
_BLOCK_COLS = 114688
_GRID = (_E_ROWS + _BLOCK_COLS - 1) // _BLOCK_COLS


def _copy_body(in_ref, out_ref):
    out_ref[...] = in_ref[...]


def kernel(embed):
    t = embed.T
    out = pl.pallas_call(
        _copy_body,
        grid=(_GRID,),
        in_specs=[pl.BlockSpec((_DIM, _BLOCK_COLS), lambda i: (0, i))],
        out_specs=pl.BlockSpec((_DIM, _BLOCK_COLS), lambda i: (0, i)),
        out_shape=jax.ShapeDtypeStruct((_DIM, _E_ROWS), jnp.float32),
    )(t)
    return out.T
